# baseline (speedup 1.0000x reference)
import functools
import math

import jax
import jax.numpy as jnp
from jax import lax
from jax.experimental import pallas as pl
from jax.experimental.pallas import tpu as pltpu
from jax.experimental.pallas import tpu_sc as plsc

D = 128
DE = 16
NOUT = 2
N_NODES = 10000
N_EDGES = 320000
G = 64

NCORES = 2
NSUB = 16
NW = NCORES * NSUB
EPW = N_EDGES // NW
CCAP = 128
CAP = NW * CCAP
TBL = N_NODES + 80


def _sc_body(ei_h, head_h, x_h,
             xs_o, code_o, geid_o, xh_o,
             table_v, dst_v, src_v, lidx_v, slot_v, srcid_v, geid_v,
             xs_st, head_v, headsh_v, xh_st, sem):
    wid = lax.axis_index("s") * NCORES + lax.axis_index("c")
    ramp = lax.iota(jnp.int32, 16)

    pltpu.sync_copy(ei_h.at[1, pl.ds(wid * EPW, EPW)], dst_v)
    pltpu.sync_copy(ei_h.at[0, pl.ds(wid * EPW, EPW)], src_v)
    pltpu.sync_copy(head_h, head_v)

    def init_tbl(i, _):
        table_v[pl.ds(i * 16, 16)] = jnp.full((16,), -1, jnp.int32)
        return 0
    lax.fori_loop(0, TBL // 16, init_tbl, 0)
    for j in range(G // 16):
        plsc.store_scatter(headsh_v, [ramp + (j * 16 + 1)],
                           head_v[pl.ds(j * 16, 16)])
    plsc.store_scatter(headsh_v, [ramp], jnp.full((16,), -1, jnp.int32),
                       mask=ramp == 0)
    for j in range(G // 16):
        cur = head_v[pl.ds(j * 16, 16)]
        prev = headsh_v[pl.ds(j * 16, 16)]
        snode = jnp.where(cur != prev, cur, N_NODES + j * 16 + ramp)
        plsc.store_scatter(table_v, [snode], ramp + j * 16)

    def init_pad(i, _):
        lidx_v[pl.ds(i * 16, 16)] = jnp.zeros((16,), jnp.int32)
        slot_v[pl.ds(i * 16, 16)] = jnp.full((16,), -1, jnp.int32)
        return 0
    lax.fori_loop(0, CCAP // 16, init_pad, 0)

    def fbody(i, cnt):
        dstv = dst_v[pl.ds(i * 16, 16)]
        slotv = plsc.load_gather(table_v, [dstv])
        mask = slotv >= 0
        mi = mask.astype(jnp.int32)
        pos = cnt + plsc.cumsum(mi) - mi
        plsc.store_scatter(lidx_v, [pos], ramp + i * 16, mask=mask)
        plsc.store_scatter(slot_v, [pos], slotv, mask=mask)
        return cnt + plsc.all_reduce_population_count(mask)
    lax.fori_loop(0, EPW // 16, fbody, jnp.zeros((16,), jnp.int32))

    def gbody(j, _):
        s = pl.ds(j * 16, 16)
        lx = lidx_v[s]
        srcid_v[s] = plsc.load_gather(src_v, [lx])
        geid_v[s] = lx + wid * EPW
        return 0
    lax.fori_loop(0, CCAP // 16, gbody, 0)

    pltpu.async_copy(x_h.at[srcid_v], xs_st, sem).wait()

    pltpu.sync_copy(xs_st, xs_o.at[wid])
    pltpu.sync_copy(slot_v.at[pl.ds(0, CCAP)],
                    code_o.at[pl.ds(wid * CCAP, CCAP)])
    pltpu.sync_copy(geid_v, geid_o.at[pl.ds(wid * CCAP, CCAP)])

    @pl.when(wid == 0)
    def _():
        pltpu.async_copy(x_h.at[head_v], xh_st, sem).wait()
        pltpu.sync_copy(xh_st, xh_o)


@functools.cache
def _sc_filter_gather():
  return pl.kernel(
    _sc_body,
    out_type=(
        jax.ShapeDtypeStruct((NW, CCAP, D), jnp.float32),
        jax.ShapeDtypeStruct((CAP,), jnp.int32),
        jax.ShapeDtypeStruct((CAP,), jnp.int32),
        jax.ShapeDtypeStruct((G, D), jnp.float32),
    ),
    mesh=plsc.VectorSubcoreMesh(
        core_axis_name="c", subcore_axis_name="s",
        num_cores=NCORES, num_subcores=NSUB),
    scratch_types=[
        pltpu.VMEM((TBL,), jnp.int32),
        pltpu.VMEM((EPW,), jnp.int32),
        pltpu.VMEM((EPW,), jnp.int32),
        pltpu.VMEM((EPW + 16,), jnp.int32),
        pltpu.VMEM((EPW + 16,), jnp.int32),
        pltpu.VMEM((CCAP,), jnp.int32),
        pltpu.VMEM((CCAP,), jnp.int32),
        pltpu.VMEM((CCAP, D), jnp.float32),
        pltpu.VMEM((G,), jnp.int32),
        pltpu.VMEM((G + 16,), jnp.int32),
        pltpu.VMEM((G, D), jnp.float32),
        pltpu.SemaphoreType.DMA,
    ],
    compiler_params=pltpu.CompilerParams(
        needs_layout_passes=False, use_tc_tiling_on_sc=False),
  )


def _scb_body(ea8_h, geid_h, ea_o, geid_v, sub_v, eaid_v, gath_v, ea_st, sem):
    wid = lax.axis_index("s") * NCORES + lax.axis_index("c")
    pltpu.sync_copy(geid_h.at[pl.ds(wid * CCAP, CCAP)], geid_v)
    for j in range(CCAP // 16):
        s = pl.ds(j * 16, 16)
        g = geid_v[s]
        eaid_v[s] = lax.shift_right_logical(g, 3)
        sub_v[s] = lax.shift_left(jnp.bitwise_and(g, 7), 4)
    pltpu.async_copy(ea8_h.at[eaid_v], gath_v, sem).wait()

    def ebody(e, _):
        sub = sub_v[pl.ds(e, 16)][0]
        ea_st[e, pl.ds(0, 16)] = gath_v[e, pl.ds(sub, 16)]
        return 0
    lax.fori_loop(0, CCAP, ebody, 0)
    pltpu.sync_copy(ea_st, ea_o.at[wid])


@functools.cache
def _sc_ea_gather():
  return pl.kernel(
    _scb_body,
    out_type=jax.ShapeDtypeStruct((NW, CCAP, D), jnp.float32),
    mesh=plsc.VectorSubcoreMesh(
        core_axis_name="c", subcore_axis_name="s",
        num_cores=NCORES, num_subcores=NSUB),
    scratch_types=[
        pltpu.VMEM((CCAP,), jnp.int32),
        pltpu.VMEM((CCAP + 16,), jnp.int32),
        pltpu.VMEM((CCAP,), jnp.int32),
        pltpu.VMEM((CCAP, D), jnp.float32),
        pltpu.VMEM((CCAP, D), jnp.float32),
        pltpu.SemaphoreType.DMA,
    ],
    compiler_params=pltpu.CompilerParams(
        needs_layout_passes=False, use_tc_tiling_on_sc=False),
  )


def tc_attention_head(xs_ref, ea_ref, cc_ref, cr_ref, xh_ref, hc_ref, hr_ref,
                      We_ref, Wq_ref, Wk_ref, Wv_ref, Wo_ref, lg_ref, lb_ref,
                      W1_ref, b1_ref, W2_ref, b2_ref, W3_ref, b3_ref, o_ref):
    f32 = jnp.float32
    dot = functools.partial(jnp.dot, preferred_element_type=f32)
    slots_c = cc_ref[...]
    slots_r = cr_ref[...]

    lane = lax.broadcasted_iota(jnp.int32, (CAP, D), 1)
    eam = jnp.where(lane < DE, ea_ref[...], 0.0)
    Wes = jnp.concatenate([We_ref[...]] * (D // DE), axis=0)
    e = dot(eam, Wes)

    xe = xs_ref[...] * e
    k = dot(xe, Wk_ref[...])
    v = dot(xe, Wv_ref[...])
    xh = xh_ref[...]
    q = dot(xh, Wq_ref[...])

    oh = (slots_c == lax.broadcasted_iota(jnp.int32, (CAP, G), 1)).astype(f32)
    ohT = (slots_r == lax.broadcasted_iota(jnp.int32, (G, CAP), 0)).astype(f32)

    qe = dot(oh, q)
    scores = jnp.sum(qe * k, axis=1, keepdims=True) * (1.0 / math.sqrt(D))
    ex = jnp.where(slots_c >= 0, jnp.exp(scores), 0.0)
    denom = dot(ohT, ex)
    dpe = dot(oh, denom)
    attn = ex / (dpe + 1e-9)
    agg = dot(ohT, attn * v)

    h = xh + dot(agg, Wo_ref[...])
    mu = jnp.mean(h, axis=1, keepdims=True)
    var = jnp.mean((h - mu) ** 2, axis=1, keepdims=True)
    hn = (h - mu) * lax.rsqrt(var + 1e-5) * lg_ref[...] + lb_ref[...]

    z = jnp.maximum(dot(hn, W1_ref[...]) + b1_ref[...], 0.0)
    z = jnp.maximum(dot(z, W2_ref[...]) + b2_ref[...], 0.0)
    outr = dot(z, W3_ref[...]) + b3_ref[...]

    eq = hc_ref[...] == hr_ref[...]
    gi = lax.broadcasted_iota(jnp.int32, (G, G), 1)
    rep = jnp.min(jnp.where(eq, gi, G), axis=1, keepdims=True)
    S = (rep == gi).astype(f32)
    o_ref[...] = dot(S, outr)


def kernel(x, edge_attr, We, Wq, Wk, Wv, Wo, ln_g, ln_b, W1, b1, W2, b2, W3,
           b3, edge_index, head_idx):
    ei = edge_index.astype(jnp.int32)
    head = head_idx.astype(jnp.int32)
    ea8 = edge_attr.reshape(N_EDGES * DE // D, D)

    xs_c, code, geid, xh = _sc_filter_gather()(ei, head, x)
    ea_c = _sc_ea_gather()(ea8, geid)

    return pl.pallas_call(
        tc_attention_head,
        out_shape=jax.ShapeDtypeStruct((G, NOUT), jnp.float32),
    )(xs_c.reshape(CAP, D), ea_c.reshape(CAP, D),
      code.reshape(CAP, 1), code.reshape(1, CAP), xh,
      head.reshape(G, 1), head.reshape(1, G),
      We, Wq, Wk, Wv, Wo, ln_g.reshape(1, D), ln_b.reshape(1, D),
      W1, b1.reshape(1, D), W2, b2.reshape(1, D), W3, b3.reshape(1, NOUT))


# speedup vs baseline: 1.3182x; 1.3182x over previous
import functools
import math

import jax
import jax.numpy as jnp
from jax import lax
from jax.experimental import pallas as pl
from jax.experimental.pallas import tpu as pltpu
from jax.experimental.pallas import tpu_sc as plsc

D = 128
DE = 16
NOUT = 2
N_NODES = 10000
N_EDGES = 320000
G = 64

NCORES = 2
NSUB = 16
NW = NCORES * NSUB
EPW = N_EDGES // NW
CCAP = 128
CAP = NW * CCAP
TBL = N_NODES + 80


def _sc_body(ei_h, head_h, x_h,
             xs_o, code_o, geid_o, xh_o,
             table_v, dst_v, src_v, lidx_v, slot_v, srcid_v, geid_v,
             xs_st, head_v, headsh_v, xh_st, sem):
    wid = lax.axis_index("s") * NCORES + lax.axis_index("c")
    ramp = lax.iota(jnp.int32, 16)

    pltpu.sync_copy(ei_h.at[1, pl.ds(wid * EPW, EPW)], dst_v)
    pltpu.sync_copy(ei_h.at[0, pl.ds(wid * EPW, EPW)], src_v)
    pltpu.sync_copy(head_h, head_v)

    def init_tbl(i, _):
        table_v[pl.ds(i * 16, 16)] = jnp.full((16,), -1, jnp.int32)
        return 0
    lax.fori_loop(0, TBL // 16, init_tbl, 0)
    for j in range(G // 16):
        plsc.store_scatter(headsh_v, [ramp + (j * 16 + 1)],
                           head_v[pl.ds(j * 16, 16)])
    plsc.store_scatter(headsh_v, [ramp], jnp.full((16,), -1, jnp.int32),
                       mask=ramp == 0)
    for j in range(G // 16):
        cur = head_v[pl.ds(j * 16, 16)]
        prev = headsh_v[pl.ds(j * 16, 16)]
        snode = jnp.where(cur != prev, cur, N_NODES + j * 16 + ramp)
        plsc.store_scatter(table_v, [snode], ramp + j * 16)

    def init_pad(i, _):
        lidx_v[pl.ds(i * 16, 16)] = jnp.zeros((16,), jnp.int32)
        slot_v[pl.ds(i * 16, 16)] = jnp.full((16,), -1, jnp.int32)
        return 0
    lax.fori_loop(0, CCAP // 16, init_pad, 0)

    def fbody(i, cnt):
        dstv = dst_v[pl.ds(i * 16, 16)]
        slotv = plsc.load_gather(table_v, [dstv])
        mask = slotv >= 0
        mi = mask.astype(jnp.int32)
        pos = cnt + plsc.cumsum(mi) - mi
        plsc.store_scatter(lidx_v, [pos], ramp + i * 16, mask=mask)
        plsc.store_scatter(slot_v, [pos], slotv, mask=mask)
        return cnt + plsc.all_reduce_population_count(mask)
    lax.fori_loop(0, EPW // 16, fbody, jnp.zeros((16,), jnp.int32))

    def gbody(j, _):
        s = pl.ds(j * 16, 16)
        lx = lidx_v[s]
        srcid_v[s] = plsc.load_gather(src_v, [lx])
        geid_v[s] = lx + wid * EPW
        return 0
    lax.fori_loop(0, CCAP // 16, gbody, 0)

    pltpu.async_copy(x_h.at[srcid_v], xs_st, sem).wait()

    pltpu.sync_copy(xs_st, xs_o.at[wid])
    pltpu.sync_copy(slot_v.at[pl.ds(0, CCAP)],
                    code_o.at[pl.ds(wid * CCAP, CCAP)])
    pltpu.sync_copy(geid_v, geid_o.at[pl.ds(wid * CCAP, CCAP)])

    @pl.when(wid == 0)
    def _():
        pltpu.async_copy(x_h.at[head_v], xh_st, sem).wait()
        pltpu.sync_copy(xh_st, xh_o)


@functools.cache
def _sc_filter_gather():
  return pl.kernel(
    _sc_body,
    out_type=(
        jax.ShapeDtypeStruct((NW, CCAP, D), jnp.float32),
        jax.ShapeDtypeStruct((CAP,), jnp.int32),
        jax.ShapeDtypeStruct((CAP,), jnp.int32),
        jax.ShapeDtypeStruct((G, D), jnp.float32),
    ),
    mesh=plsc.VectorSubcoreMesh(
        core_axis_name="c", subcore_axis_name="s",
        num_cores=NCORES, num_subcores=NSUB),
    scratch_types=[
        pltpu.VMEM((TBL,), jnp.int32),
        pltpu.VMEM((EPW,), jnp.int32),
        pltpu.VMEM((EPW,), jnp.int32),
        pltpu.VMEM((EPW + 16,), jnp.int32),
        pltpu.VMEM((EPW + 16,), jnp.int32),
        pltpu.VMEM((CCAP,), jnp.int32),
        pltpu.VMEM((CCAP,), jnp.int32),
        pltpu.VMEM((CCAP, D), jnp.float32),
        pltpu.VMEM((G,), jnp.int32),
        pltpu.VMEM((G + 16,), jnp.int32),
        pltpu.VMEM((G, D), jnp.float32),
        pltpu.SemaphoreType.DMA,
    ],
    compiler_params=pltpu.CompilerParams(
        needs_layout_passes=False, use_tc_tiling_on_sc=False),
  )


def _scb_body(ea_h, geid_h, ea_o, geid_v, ea_st, sem):
    wid = lax.axis_index("s") * NCORES + lax.axis_index("c")
    pltpu.sync_copy(geid_h.at[pl.ds(wid * CCAP, CCAP)], geid_v.at[pl.ds(0, CCAP)])

    def fire(e, _):
        g = geid_v[pl.ds(e, 16)][0]
        pltpu.async_copy(ea_h.at[g], ea_st.at[e], sem)
        return 0
    lax.fori_loop(0, CCAP, fire, 0)

    def drain(e, _):
        g = geid_v[pl.ds(e, 16)][0]
        pltpu.make_async_copy(ea_h.at[g], ea_st.at[e], sem).wait()
        return 0
    lax.fori_loop(0, CCAP, drain, 0)
    pltpu.sync_copy(ea_st, ea_o.at[wid])


@functools.cache
def _sc_ea_gather():
  return pl.kernel(
    _scb_body,
    out_type=jax.ShapeDtypeStruct((NW, CCAP, DE), jnp.float32),
    mesh=plsc.VectorSubcoreMesh(
        core_axis_name="c", subcore_axis_name="s",
        num_cores=NCORES, num_subcores=NSUB),
    scratch_types=[
        pltpu.VMEM((CCAP + 16,), jnp.int32),
        pltpu.VMEM((CCAP, DE), jnp.float32),
        pltpu.SemaphoreType.DMA,
    ],
    compiler_params=pltpu.CompilerParams(needs_layout_passes=False),
  )


def tc_attention_head(xs_ref, ea_ref, cc_ref, cr_ref, xh_ref, hc_ref, hr_ref,
                      We_ref, Wq_ref, Wk_ref, Wv_ref, Wo_ref, lg_ref, lb_ref,
                      W1_ref, b1_ref, W2_ref, b2_ref, W3_ref, b3_ref, o_ref):
    f32 = jnp.float32
    dot = functools.partial(jnp.dot, preferred_element_type=f32)
    slots_c = cc_ref[...]
    slots_r = cr_ref[...]

    e = dot(ea_ref[...], We_ref[...])

    xe = xs_ref[...] * e
    k = dot(xe, Wk_ref[...])
    v = dot(xe, Wv_ref[...])
    xh = xh_ref[...]
    q = dot(xh, Wq_ref[...])

    oh = (slots_c == lax.broadcasted_iota(jnp.int32, (CAP, G), 1)).astype(f32)
    ohT = (slots_r == lax.broadcasted_iota(jnp.int32, (G, CAP), 0)).astype(f32)

    qe = dot(oh, q)
    scores = jnp.sum(qe * k, axis=1, keepdims=True) * (1.0 / math.sqrt(D))
    ex = jnp.where(slots_c >= 0, jnp.exp(scores), 0.0)
    denom = dot(ohT, ex)
    dpe = dot(oh, denom)
    attn = ex / (dpe + 1e-9)
    agg = dot(ohT, attn * v)

    h = xh + dot(agg, Wo_ref[...])
    mu = jnp.mean(h, axis=1, keepdims=True)
    var = jnp.mean((h - mu) ** 2, axis=1, keepdims=True)
    hn = (h - mu) * lax.rsqrt(var + 1e-5) * lg_ref[...] + lb_ref[...]

    z = jnp.maximum(dot(hn, W1_ref[...]) + b1_ref[...], 0.0)
    z = jnp.maximum(dot(z, W2_ref[...]) + b2_ref[...], 0.0)
    outr = dot(z, W3_ref[...]) + b3_ref[...]

    eq = hc_ref[...] == hr_ref[...]
    gi = lax.broadcasted_iota(jnp.int32, (G, G), 1)
    rep = jnp.min(jnp.where(eq, gi, G), axis=1, keepdims=True)
    S = (rep == gi).astype(f32)
    o_ref[...] = dot(S, outr)


def kernel(x, edge_attr, We, Wq, Wk, Wv, Wo, ln_g, ln_b, W1, b1, W2, b2, W3,
           b3, edge_index, head_idx):
    ei = edge_index.astype(jnp.int32)
    head = head_idx.astype(jnp.int32)
    xs_c, code, geid, xh = _sc_filter_gather()(ei, head, x)
    ea_c = _sc_ea_gather()(edge_attr, geid)

    return pl.pallas_call(
        tc_attention_head,
        out_shape=jax.ShapeDtypeStruct((G, NOUT), jnp.float32),
    )(xs_c.reshape(CAP, D), ea_c.reshape(CAP, DE),
      code.reshape(CAP, 1), code.reshape(1, CAP), xh,
      head.reshape(G, 1), head.reshape(1, G),
      We, Wq, Wk, Wv, Wo, ln_g.reshape(1, D), ln_b.reshape(1, D),
      W1, b1.reshape(1, D), W2, b2.reshape(1, D), W3, b3.reshape(1, NOUT))
